# baseline (speedup 1.0000x reference)
import functools

import jax
import jax.numpy as jnp
from jax import lax
from jax.experimental import pallas as pl
from jax.experimental.pallas import tpu as pltpu
from jax.experimental.pallas import tpu_sc as plsc

_NUM_CORES = 2
_NUM_SUBCORES = 16
_NUM_WORKERS = _NUM_CORES * _NUM_SUBCORES
_CHUNK = 128
_WINDOW = 16


def kernel(users, user_embedding):
    B = users.shape[0]
    V, D = user_embedding.shape
    b_per_w = B // _NUM_WORKERS
    n_idx = b_per_w * D
    n_g = n_idx // _CHUNK

    mesh = plsc.VectorSubcoreMesh(
        core_axis_name="c", subcore_axis_name="s",
        num_cores=_NUM_CORES, num_subcores=_NUM_SUBCORES)

    @functools.partial(
        pl.kernel,
        mesh=mesh,
        out_type=jax.ShapeDtypeStruct((D, B), jnp.float32),
        scratch_types=[
            pltpu.VMEM((b_per_w,), jnp.int32),
            pltpu.VMEM((n_idx,), jnp.int32),
            pltpu.VMEM((n_idx,), jnp.float32),
            pltpu.SemaphoreType.DMA,
        ],
        compiler_params=pltpu.CompilerParams(use_tc_tiling_on_sc=False),
    )
    def gather_kernel(idx_hbm, tflat_hbm, out_hbm, idx_v, widx_v, buf_v, sem):
        wid = lax.axis_index("s") * _NUM_CORES + lax.axis_index("c")
        base = wid * b_per_w
        pltpu.sync_copy(idx_hbm.at[wid], idx_v)

        def build(k, carry):
            uv = idx_v[pl.ds(k * 16, 16)]
            for f in range(D):
                widx_v[pl.ds(f * b_per_w + k * 16, 16)] = uv + f * V
            return carry

        lax.fori_loop(0, b_per_w // 16, build, 0)

        def fire(g, carry):
            pltpu.async_copy(
                tflat_hbm.at[widx_v.at[pl.ds(g * _CHUNK, _CHUNK)]],
                buf_v.at[pl.ds(g * _CHUNK, _CHUNK)],
                sem)

            @pl.when(g >= _WINDOW)
            def _drain():
                pltpu.make_async_copy(
                    tflat_hbm.at[pl.ds(0, _CHUNK)],
                    buf_v.at[pl.ds((g - _WINDOW) * _CHUNK, _CHUNK)],
                    sem).wait()

            return carry

        lax.fori_loop(0, n_g, fire, 0)
        pltpu.make_async_copy(
            tflat_hbm.at[pl.ds(0, _WINDOW * _CHUNK)],
            buf_v.at[pl.ds((n_g - _WINDOW) * _CHUNK, _WINDOW * _CHUNK)],
            sem).wait()
        for f in range(D):
            pltpu.sync_copy(
                buf_v.at[pl.ds(f * b_per_w, b_per_w)],
                out_hbm.at[f, pl.ds(base, b_per_w)])

    idx = users.astype(jnp.int32).reshape(_NUM_WORKERS, b_per_w)
    out_t = gather_kernel(idx, user_embedding.T.ravel())
    return out_t.T


# speedup vs baseline: 1.4597x; 1.4597x over previous
import functools

import jax
import jax.numpy as jnp
from jax import lax
from jax.experimental import pallas as pl
from jax.experimental.pallas import tpu as pltpu
from jax.experimental.pallas import tpu_sc as plsc

_NUM_CORES = 2
_NUM_SUBCORES = 16
_NUM_WORKERS = _NUM_CORES * _NUM_SUBCORES
_CHUNK = 128
_WINDOW = 16


def kernel(users, user_embedding):
    B = users.shape[0]
    V, D = user_embedding.shape
    b_per_w = B // _NUM_WORKERS
    n_idx = b_per_w * D
    n_g = n_idx // _CHUNK

    mesh = plsc.VectorSubcoreMesh(
        core_axis_name="c", subcore_axis_name="s",
        num_cores=_NUM_CORES, num_subcores=_NUM_SUBCORES)

    @functools.partial(
        pl.kernel,
        mesh=mesh,
        out_type=jax.ShapeDtypeStruct((D, B), jnp.float32),
        scratch_types=[
            pltpu.VMEM((b_per_w,), jnp.int32),
            pltpu.VMEM((n_idx,), jnp.int32),
            pltpu.VMEM((n_idx,), jnp.float32),
            pltpu.SemaphoreType.DMA,
        ],
        compiler_params=pltpu.CompilerParams(use_tc_tiling_on_sc=False),
    )
    def gather_kernel(idx_hbm, tflat_hbm, out_hbm, idx_v, widx_v, buf_v, sem):
        wid = lax.axis_index("s") * _NUM_CORES + lax.axis_index("c")
        base = wid * b_per_w
        pltpu.sync_copy(idx_hbm.at[wid], idx_v)

        def build(k, carry):
            uv = idx_v[pl.ds(k * 16, 16)]
            for f in range(D):
                widx_v[pl.ds(f * b_per_w + k * 16, 16)] = uv + f * V
            return carry

        lax.fori_loop(0, b_per_w // 16, build, 0)

        def fire(g, carry):
            pltpu.async_copy(
                tflat_hbm.at[widx_v.at[pl.ds(g * _CHUNK, _CHUNK)]],
                buf_v.at[pl.ds(g * _CHUNK, _CHUNK)],
                sem)

            @pl.when(g >= _WINDOW)
            def _drain():
                pltpu.make_async_copy(
                    tflat_hbm.at[pl.ds(0, _CHUNK)],
                    buf_v.at[pl.ds((g - _WINDOW) * _CHUNK, _CHUNK)],
                    sem).wait()

            return carry

        lax.fori_loop(0, n_g, fire, 0)
        pltpu.make_async_copy(
            tflat_hbm.at[pl.ds(0, _WINDOW * _CHUNK)],
            buf_v.at[pl.ds((n_g - _WINDOW) * _CHUNK, _WINDOW * _CHUNK)],
            sem).wait()
        for f in range(D):
            pltpu.sync_copy(
                buf_v.at[pl.ds(f * b_per_w, b_per_w)],
                out_hbm.at[f, pl.ds(base, b_per_w)])

    idx = users.astype(jnp.int32).reshape(_NUM_WORKERS, b_per_w)
    tflat = jnp.concatenate([user_embedding[:, f] for f in range(D)])
    out_t = gather_kernel(idx, tflat)
    return out_t.T


# speedup vs baseline: 4.9463x; 3.3886x over previous
import functools

import jax
import jax.numpy as jnp
from jax import lax
from jax.experimental import pallas as pl
from jax.experimental.pallas import tpu as pltpu
from jax.experimental.pallas import tpu_sc as plsc

_NUM_CORES = 2
_NUM_SUBCORES = 16
_NUM_WORKERS = _NUM_CORES * _NUM_SUBCORES
_CHUNK = 128


def kernel(users, user_embedding):
    B = users.shape[0]
    V, D = user_embedding.shape
    b_per_w = B // _NUM_WORKERS
    n_chunks = b_per_w // _CHUNK

    mesh = plsc.VectorSubcoreMesh(
        core_axis_name="c", subcore_axis_name="s",
        num_cores=_NUM_CORES, num_subcores=_NUM_SUBCORES)

    @functools.partial(
        pl.kernel,
        mesh=mesh,
        out_type=jax.ShapeDtypeStruct((B, D), jnp.float32),
        scratch_types=[
            pltpu.VMEM((n_chunks, _CHUNK), jnp.int32),
            pltpu.VMEM((b_per_w, D), jnp.float32),
            pltpu.SemaphoreType.DMA,
        ],
        compiler_params=pltpu.CompilerParams(use_tc_tiling_on_sc=False),
    )
    def gather_kernel(idx_hbm, table_hbm, out_hbm, idx_v, rows_v, sem):
        wid = lax.axis_index("s") * _NUM_CORES + lax.axis_index("c")
        base = wid * b_per_w
        pltpu.sync_copy(idx_hbm.at[wid], idx_v)
        copies = []
        for j in range(n_chunks):
            copies.append(pltpu.async_copy(
                table_hbm.at[idx_v.at[j]],
                rows_v.at[pl.ds(j * _CHUNK, _CHUNK)],
                sem))
        for c in copies:
            c.wait()
        pltpu.sync_copy(rows_v, out_hbm.at[pl.ds(base, b_per_w)])

    idx = users.astype(jnp.int32).reshape(_NUM_WORKERS, n_chunks, _CHUNK)
    return gather_kernel(idx, user_embedding)
